# initial kernel scaffold (unmeasured)
import jax
import jax.numpy as jnp
from jax import lax
from jax.experimental import pallas as pl
from jax.experimental.pallas import tpu as pltpu

N_DEV = 4
HPD = 8
DH = 128
SCALE = 0.08838834764831843
BLK = 64


def kernel(x, Wq, K_ext, V_ext, Wo):
    my = lax.axis_index("i")
    Sq, D = x.shape[1], x.shape[2]

    xb = x[0].astype(jnp.bfloat16)
    wq = Wq.astype(jnp.bfloat16)
    wo = Wo.astype(jnp.bfloat16)
    k_my = jnp.transpose(
        lax.dynamic_index_in_dim(K_ext, my, 0, keepdims=False), (1, 0, 2)
    ).astype(jnp.bfloat16)
    v_my = jnp.transpose(
        lax.dynamic_index_in_dim(V_ext, my, 0, keepdims=False), (1, 0, 2)
    ).astype(jnp.bfloat16)

    def body(x_ref, wq_ref, k_ref, v_ref, wo_ref, out_ref,
             comm_wq, comm_wo, ctx_ref,
             wq_send, wq_recv, wo_send, wo_recv):
        my_pos = lax.axis_index("i")
        left = lax.rem(my_pos + N_DEV - 1, N_DEV)
        right = lax.rem(my_pos + 1, N_DEV)

        barrier = pltpu.get_barrier_semaphore()
        for nbr in (left, right):
            pl.semaphore_signal(
                barrier, inc=1, device_id=(nbr,),
                device_id_type=pl.DeviceIdType.MESH,
            )
        pl.semaphore_wait(barrier, 2)

        qb = lax.broadcasted_iota(jnp.int32, (Sq, Sq), 0) // BLK
        kb = lax.broadcasted_iota(jnp.int32, (Sq, Sq), 1) // BLK
        allow = kb <= qb

        for s in range(N_DEV):
            wq_s = wq_ref if s == 0 else comm_wq.at[s - 1]
            wo_s = wo_ref if s == 0 else comm_wo.at[s - 1]
            if s < N_DEV - 1:
                rq = pltpu.make_async_remote_copy(
                    src_ref=wq_s, dst_ref=comm_wq.at[s],
                    send_sem=wq_send.at[s], recv_sem=wq_recv.at[s],
                    device_id=(right,), device_id_type=pl.DeviceIdType.MESH,
                )
                rq.start()
                ro = pltpu.make_async_remote_copy(
                    src_ref=wo_s, dst_ref=comm_wo.at[s],
                    send_sem=wo_send.at[s], recv_sem=wo_recv.at[s],
                    device_id=(right,), device_id_type=pl.DeviceIdType.MESH,
                )
                ro.start()

            g = lax.rem(my_pos - s + N_DEV, N_DEV)
            q_all = jnp.dot(
                x_ref[...], wq_s[...], preferred_element_type=jnp.float32
            ).astype(jnp.bfloat16)

            def head_body(h, carry):
                head = g * HPD + h
                q = lax.dynamic_slice(q_all, (0, h * DH), (Sq, DH))
                k = k_ref[head]
                sc = lax.dot_general(
                    q, k, (((1,), (1,)), ((), ())),
                    preferred_element_type=jnp.float32,
                ) * SCALE
                sc = jnp.where(allow, sc, -1e9)
                m = jnp.max(sc, axis=1, keepdims=True)
                w = jnp.exp(sc - m)
                p = (w / jnp.sum(w, axis=1, keepdims=True)).astype(jnp.bfloat16)
                ctx = jnp.dot(p, v_ref[head], preferred_element_type=jnp.float32)
                ctx_ref[:, pl.ds(h * DH, DH)] = ctx.astype(jnp.bfloat16)
                return carry

            lax.fori_loop(0, HPD, head_body, 0)
            part = jnp.dot(
                ctx_ref[...], wo_s[...], preferred_element_type=jnp.float32
            )
            if s == 0:
                out_ref[0] = part
            else:
                out_ref[0] += part

            if s < N_DEV - 1:
                rq.wait()
                ro.wait()

    return pl.pallas_call(
        body,
        out_shape=jax.ShapeDtypeStruct((1, Sq, D), jnp.float32),
        in_specs=[pl.BlockSpec(memory_space=pltpu.VMEM)] * 5,
        out_specs=pl.BlockSpec(memory_space=pltpu.VMEM),
        scratch_shapes=[
            pltpu.VMEM((N_DEV - 1, D, HPD * DH), jnp.bfloat16),
            pltpu.VMEM((N_DEV - 1, HPD * DH, D), jnp.bfloat16),
            pltpu.VMEM((Sq, HPD * DH), jnp.bfloat16),
            pltpu.SemaphoreType.DMA((N_DEV - 1,)),
            pltpu.SemaphoreType.DMA((N_DEV - 1,)),
            pltpu.SemaphoreType.DMA((N_DEV - 1,)),
            pltpu.SemaphoreType.DMA((N_DEV - 1,)),
        ],
        compiler_params=pltpu.CompilerParams(collective_id=0),
    )(xb, wq, k_my, v_my, wo)


# baseline (device time: 210189 ns/iter reference)
import jax
import jax.numpy as jnp
from jax import lax
from jax.experimental import pallas as pl
from jax.experimental.pallas import tpu as pltpu

N_DEV = 4
HPD = 8
DH = 128
SCALE = 0.08838834764831843
BLK = 64


def kernel(x, Wq, K_ext, V_ext, Wo):
    my = lax.axis_index("i")
    Sq, D = x.shape[1], x.shape[2]

    xb = x[0].astype(jnp.bfloat16)
    wq = Wq.astype(jnp.bfloat16)
    wo = Wo.astype(jnp.bfloat16)
    k_my = jnp.transpose(
        lax.dynamic_index_in_dim(K_ext, my, 0, keepdims=False), (1, 0, 2)
    ).astype(jnp.bfloat16)
    v_my = jnp.transpose(
        lax.dynamic_index_in_dim(V_ext, my, 0, keepdims=False), (1, 0, 2)
    ).astype(jnp.bfloat16)

    def body(x_ref, wq_ref, k_ref, v_ref, wo_ref, out_ref,
             comm_wq, comm_wo, ctx_ref, q_ref,
             wq_send, wq_recv, wo_send, wo_recv):
        my_pos = lax.axis_index("i")
        left = lax.rem(my_pos + N_DEV - 1, N_DEV)
        right = lax.rem(my_pos + 1, N_DEV)

        barrier = pltpu.get_barrier_semaphore()
        for nbr in (left, right):
            pl.semaphore_signal(
                barrier, inc=1, device_id=(nbr,),
                device_id_type=pl.DeviceIdType.MESH,
            )
        pl.semaphore_wait(barrier, 2)

        qb = lax.broadcasted_iota(jnp.int32, (Sq, Sq), 0) // BLK
        kb = lax.broadcasted_iota(jnp.int32, (Sq, Sq), 1) // BLK
        allow = kb <= qb

        for s in range(N_DEV):
            wq_s = wq_ref if s == 0 else comm_wq.at[s - 1]
            wo_s = wo_ref if s == 0 else comm_wo.at[s - 1]
            if s < N_DEV - 1:
                rq = pltpu.make_async_remote_copy(
                    src_ref=wq_s, dst_ref=comm_wq.at[s],
                    send_sem=wq_send.at[s], recv_sem=wq_recv.at[s],
                    device_id=(right,), device_id_type=pl.DeviceIdType.MESH,
                )
                rq.start()
                ro = pltpu.make_async_remote_copy(
                    src_ref=wo_s, dst_ref=comm_wo.at[s],
                    send_sem=wo_send.at[s], recv_sem=wo_recv.at[s],
                    device_id=(right,), device_id_type=pl.DeviceIdType.MESH,
                )
                ro.start()

            g = lax.rem(my_pos - s + N_DEV, N_DEV)
            q_ref[...] = jnp.dot(
                x_ref[...], wq_s[...], preferred_element_type=jnp.float32
            ).astype(jnp.bfloat16)

            def head_body(h, carry):
                head = g * HPD + h
                q = q_ref[:, pl.ds(h * DH, DH)]
                k = k_ref[head]
                sc = lax.dot_general(
                    q, k, (((1,), (1,)), ((), ())),
                    preferred_element_type=jnp.float32,
                ) * SCALE
                sc = jnp.where(allow, sc, -1e9)
                m = jnp.max(sc, axis=1, keepdims=True)
                w = jnp.exp(sc - m)
                p = (w / jnp.sum(w, axis=1, keepdims=True)).astype(jnp.bfloat16)
                ctx = jnp.dot(p, v_ref[head], preferred_element_type=jnp.float32)
                ctx_ref[:, pl.ds(h * DH, DH)] = ctx.astype(jnp.bfloat16)
                return carry

            lax.fori_loop(0, HPD, head_body, 0)
            part = jnp.dot(
                ctx_ref[...], wo_s[...], preferred_element_type=jnp.float32
            )
            if s == 0:
                out_ref[0] = part
            else:
                out_ref[0] += part

            if s < N_DEV - 1:
                rq.wait()
                ro.wait()

    return pl.pallas_call(
        body,
        out_shape=jax.ShapeDtypeStruct((1, Sq, D), jnp.float32),
        in_specs=[pl.BlockSpec(memory_space=pltpu.VMEM)] * 5,
        out_specs=pl.BlockSpec(memory_space=pltpu.VMEM),
        scratch_shapes=[
            pltpu.VMEM((N_DEV - 1, D, HPD * DH), jnp.bfloat16),
            pltpu.VMEM((N_DEV - 1, HPD * DH, D), jnp.bfloat16),
            pltpu.VMEM((Sq, HPD * DH), jnp.bfloat16),
            pltpu.VMEM((Sq, HPD * DH), jnp.bfloat16),
            pltpu.SemaphoreType.DMA((N_DEV - 1,)),
            pltpu.SemaphoreType.DMA((N_DEV - 1,)),
            pltpu.SemaphoreType.DMA((N_DEV - 1,)),
            pltpu.SemaphoreType.DMA((N_DEV - 1,)),
        ],
        compiler_params=pltpu.CompilerParams(collective_id=0),
    )(xb, wq, k_my, v_my, wo)


# device time: 137253 ns/iter; 1.5314x vs baseline; 1.5314x over previous
import jax
import jax.numpy as jnp
from jax import lax
from jax.experimental import pallas as pl
from jax.experimental.pallas import tpu as pltpu

N_DEV = 4
HPD = 8
HH = HPD // 2
DH = 128
SCALE = 0.08838834764831843
BLK = 64


def kernel(x, Wq, K_ext, V_ext, Wo):
    my = lax.axis_index("i")
    Sq, D = x.shape[1], x.shape[2]
    HW = HH * DH

    xb = x[0].astype(jnp.bfloat16)
    wqa = Wq[:, :HW].astype(jnp.bfloat16)
    wqb = Wq[:, HW:].astype(jnp.bfloat16)
    woa = Wo[:HW, :].astype(jnp.bfloat16)
    wob = Wo[HW:, :].astype(jnp.bfloat16)
    k_my = jnp.transpose(
        lax.dynamic_index_in_dim(K_ext, my, 0, keepdims=False), (1, 0, 2)
    ).astype(jnp.bfloat16)
    v_my = jnp.transpose(
        lax.dynamic_index_in_dim(V_ext, my, 0, keepdims=False), (1, 0, 2)
    ).astype(jnp.bfloat16)

    def body(x_ref, wqa_ref, wqb_ref, woa_ref, wob_ref, k_ref, v_ref,
             out_ref,
             comm_wqa, comm_woa, comm_wqb, comm_wob,
             qa_ref, qb_ref, ctxa_ref, ctxb_ref,
             sa_q, ra_q, sa_o, ra_o, sb_q, rb_q, sb_o, rb_o):
        my_pos = lax.axis_index("i")
        left = lax.rem(my_pos + N_DEV - 1, N_DEV)
        right = lax.rem(my_pos + 1, N_DEV)

        barrier = pltpu.get_barrier_semaphore()
        for nbr in (left, right):
            pl.semaphore_signal(
                barrier, inc=1, device_id=(nbr,),
                device_id_type=pl.DeviceIdType.MESH,
            )
        pl.semaphore_wait(barrier, 2)

        qblk = lax.broadcasted_iota(jnp.int32, (Sq, Sq), 0) // BLK
        kblk = lax.broadcasted_iota(jnp.int32, (Sq, Sq), 1) // BLK
        allow = kblk <= qblk

        def rc(src, dst, ssem, rsem, dev):
            return pltpu.make_async_remote_copy(
                src_ref=src, dst_ref=dst, send_sem=ssem, recv_sem=rsem,
                device_id=(dev,), device_id_type=pl.DeviceIdType.MESH,
            )

        for s in range(N_DEV):
            wqa_s = wqa_ref if s == 0 else comm_wqa.at[s - 1]
            woa_s = woa_ref if s == 0 else comm_woa.at[s - 1]
            wqb_s = wqb_ref if s == 0 else comm_wqb.at[s - 1]
            wob_s = wob_ref if s == 0 else comm_wob.at[s - 1]
            if s < N_DEV - 1:
                rdmas = [
                    rc(wqa_s, comm_wqa.at[s], sa_q.at[s], ra_q.at[s], right),
                    rc(woa_s, comm_woa.at[s], sa_o.at[s], ra_o.at[s], right),
                    rc(wqb_s, comm_wqb.at[s], sb_q.at[s], rb_q.at[s], left),
                    rc(wob_s, comm_wob.at[s], sb_o.at[s], rb_o.at[s], left),
                ]
                for r in rdmas:
                    r.start()

            gr = lax.rem(my_pos - s + N_DEV, N_DEV)
            gl = lax.rem(my_pos + s, N_DEV)
            qa_ref[...] = jnp.dot(
                x_ref[...], wqa_s[...], preferred_element_type=jnp.float32
            ).astype(jnp.bfloat16)
            qb_ref[...] = jnp.dot(
                x_ref[...], wqb_s[...], preferred_element_type=jnp.float32
            ).astype(jnp.bfloat16)

            def one_head(q_ref, ctx_ref, h, head):
                q = q_ref[:, pl.ds(h * DH, DH)]
                sc = lax.dot_general(
                    q, k_ref[head], (((1,), (1,)), ((), ())),
                    preferred_element_type=jnp.float32,
                ) * SCALE
                w = jnp.exp(jnp.where(allow, sc, -1e9))
                p = (w / jnp.sum(w, axis=1, keepdims=True)).astype(jnp.bfloat16)
                ctx = jnp.dot(p, v_ref[head], preferred_element_type=jnp.float32)
                ctx_ref[:, pl.ds(h * DH, DH)] = ctx.astype(jnp.bfloat16)

            def head_body(h, carry):
                one_head(qa_ref, ctxa_ref, h, gr * HPD + h)
                one_head(qb_ref, ctxb_ref, h, gl * HPD + HH + h)
                return carry

            lax.fori_loop(0, HH, head_body, 0)
            part = jnp.dot(
                ctxa_ref[...], woa_s[...], preferred_element_type=jnp.float32
            ) + jnp.dot(
                ctxb_ref[...], wob_s[...], preferred_element_type=jnp.float32
            )
            if s == 0:
                out_ref[0] = part
            else:
                out_ref[0] += part

            if s < N_DEV - 1:
                for r in rdmas:
                    r.wait()

    nh = N_DEV - 1
    return pl.pallas_call(
        body,
        out_shape=jax.ShapeDtypeStruct((1, Sq, D), jnp.float32),
        in_specs=[pl.BlockSpec(memory_space=pltpu.VMEM)] * 7,
        out_specs=pl.BlockSpec(memory_space=pltpu.VMEM),
        scratch_shapes=[
            pltpu.VMEM((nh, D, HW), jnp.bfloat16),
            pltpu.VMEM((nh, HW, D), jnp.bfloat16),
            pltpu.VMEM((nh, D, HW), jnp.bfloat16),
            pltpu.VMEM((nh, HW, D), jnp.bfloat16),
            pltpu.VMEM((Sq, HW), jnp.bfloat16),
            pltpu.VMEM((Sq, HW), jnp.bfloat16),
            pltpu.VMEM((Sq, HW), jnp.bfloat16),
            pltpu.VMEM((Sq, HW), jnp.bfloat16),
        ] + [pltpu.SemaphoreType.DMA((nh,))] * 8,
        compiler_params=pltpu.CompilerParams(collective_id=0),
    )(xb, wqa, wqb, woa, wob, k_my, v_my)


# device time: 134257 ns/iter; 1.5656x vs baseline; 1.0223x over previous
import jax
import jax.numpy as jnp
from jax import lax
from jax.experimental import pallas as pl
from jax.experimental.pallas import tpu as pltpu

N_DEV = 4
HPD = 8
HH = HPD // 2
DH = 128
SCALE = 0.08838834764831843
BLK = 64
QT = 256


def kernel(x, Wq, K_ext, V_ext, Wo):
    my = lax.axis_index("i")
    Sq, D = x.shape[1], x.shape[2]
    HW = HH * DH

    xb = x[0].astype(jnp.bfloat16)
    wqa = Wq[:, :HW].astype(jnp.bfloat16)
    wqb = Wq[:, HW:].astype(jnp.bfloat16)
    woa = Wo[:HW, :].astype(jnp.bfloat16)
    wob = Wo[HW:, :].astype(jnp.bfloat16)
    k_my = jnp.transpose(
        lax.dynamic_index_in_dim(K_ext, my, 0, keepdims=False), (1, 0, 2)
    ).astype(jnp.bfloat16)
    v_my = jnp.transpose(
        lax.dynamic_index_in_dim(V_ext, my, 0, keepdims=False), (1, 0, 2)
    ).astype(jnp.bfloat16)

    def body(x_ref, wqa_ref, wqb_ref, woa_ref, wob_ref, k_ref, v_ref,
             out_ref,
             comm_wqa, comm_woa, comm_wqb, comm_wob,
             qa_ref, qb_ref, ctxa_ref, ctxb_ref,
             sa_q, ra_q, sa_o, ra_o, sb_q, rb_q, sb_o, rb_o):
        my_pos = lax.axis_index("i")
        left = lax.rem(my_pos + N_DEV - 1, N_DEV)
        right = lax.rem(my_pos + 1, N_DEV)

        barrier = pltpu.get_barrier_semaphore()
        for nbr in (left, right):
            pl.semaphore_signal(
                barrier, inc=1, device_id=(nbr,),
                device_id_type=pl.DeviceIdType.MESH,
            )
        pl.semaphore_wait(barrier, 2)

        def rc(src, dst, ssem, rsem, dev):
            return pltpu.make_async_remote_copy(
                src_ref=src, dst_ref=dst, send_sem=ssem, recv_sem=rsem,
                device_id=(dev,), device_id_type=pl.DeviceIdType.MESH,
            )

        for s in range(N_DEV):
            wqa_s = wqa_ref if s == 0 else comm_wqa.at[s - 1]
            woa_s = woa_ref if s == 0 else comm_woa.at[s - 1]
            wqb_s = wqb_ref if s == 0 else comm_wqb.at[s - 1]
            wob_s = wob_ref if s == 0 else comm_wob.at[s - 1]
            if s < N_DEV - 1:
                rdmas = [
                    rc(wqa_s, comm_wqa.at[s], sa_q.at[s], ra_q.at[s], right),
                    rc(woa_s, comm_woa.at[s], sa_o.at[s], ra_o.at[s], right),
                    rc(wqb_s, comm_wqb.at[s], sb_q.at[s], rb_q.at[s], left),
                    rc(wob_s, comm_wob.at[s], sb_o.at[s], rb_o.at[s], left),
                ]
                for r in rdmas:
                    r.start()

            gr = lax.rem(my_pos - s + N_DEV, N_DEV)
            gl = lax.rem(my_pos + s, N_DEV)
            qa_ref[...] = jnp.dot(
                x_ref[...], wqa_s[...], preferred_element_type=jnp.float32
            ).astype(jnp.bfloat16)
            qb_ref[...] = jnp.dot(
                x_ref[...], wqb_s[...], preferred_element_type=jnp.float32
            ).astype(jnp.bfloat16)

            def one_head(q_ref, ctx_ref, h, head, t):
                r0 = t * QT
                kend = (t + 1) * QT
                q = q_ref[pl.ds(r0, QT), pl.ds(h * DH, DH)]
                k = k_ref[head, pl.ds(0, kend), :]
                sc = lax.dot_general(
                    q, k, (((1,), (1,)), ((), ())),
                    preferred_element_type=jnp.float32,
                ) * SCALE
                qblk = (lax.broadcasted_iota(jnp.int32, (QT, kend), 0) + r0) // BLK
                kblk = lax.broadcasted_iota(jnp.int32, (QT, kend), 1) // BLK
                w = jnp.exp(jnp.where(kblk <= qblk, sc, -1e9))
                denom = jnp.sum(w, axis=1, keepdims=True)
                ctx = jnp.dot(
                    w.astype(jnp.bfloat16), v_ref[head, pl.ds(0, kend), :],
                    preferred_element_type=jnp.float32,
                )
                ctx_ref[pl.ds(r0, QT), pl.ds(h * DH, DH)] = (
                    ctx * (1.0 / denom)
                ).astype(jnp.bfloat16)

            for t in range(Sq // QT):
                def head_body(h, carry):
                    one_head(qa_ref, ctxa_ref, h, gr * HPD + h, t)
                    one_head(qb_ref, ctxb_ref, h, gl * HPD + HH + h, t)
                    return carry

                lax.fori_loop(0, HH, head_body, 0)
            part = jnp.dot(
                ctxa_ref[...], woa_s[...], preferred_element_type=jnp.float32
            ) + jnp.dot(
                ctxb_ref[...], wob_s[...], preferred_element_type=jnp.float32
            )
            if s == 0:
                out_ref[0] = part
            else:
                out_ref[0] += part

            if s < N_DEV - 1:
                for r in rdmas:
                    r.wait()

    nh = N_DEV - 1
    return pl.pallas_call(
        body,
        out_shape=jax.ShapeDtypeStruct((1, Sq, D), jnp.float32),
        in_specs=[pl.BlockSpec(memory_space=pltpu.VMEM)] * 7,
        out_specs=pl.BlockSpec(memory_space=pltpu.VMEM),
        scratch_shapes=[
            pltpu.VMEM((nh, D, HW), jnp.bfloat16),
            pltpu.VMEM((nh, HW, D), jnp.bfloat16),
            pltpu.VMEM((nh, D, HW), jnp.bfloat16),
            pltpu.VMEM((nh, HW, D), jnp.bfloat16),
            pltpu.VMEM((Sq, HW), jnp.bfloat16),
            pltpu.VMEM((Sq, HW), jnp.bfloat16),
            pltpu.VMEM((Sq, HW), jnp.bfloat16),
            pltpu.VMEM((Sq, HW), jnp.bfloat16),
        ] + [pltpu.SemaphoreType.DMA((nh,))] * 8,
        compiler_params=pltpu.CompilerParams(collective_id=0),
    )(xb, wqa, wqb, woa, wob, k_my, v_my)
